# Initial kernel scaffold; baseline (speedup 1.0000x reference)
import jax
import jax.numpy as jnp
from jax.experimental import pallas as pl


def kernel(h, x, edge_index, params):
    raise NotImplementedError("write your pallas kernel here")


# speedup vs baseline: 1.9313x; 1.9313x over previous
import functools

import jax
import jax.numpy as jnp
from jax import lax
from jax.experimental import pallas as pl
from jax.experimental.pallas import tpu as pltpu
from jax.experimental.pallas import tpu_sc as plsc

_f32 = jnp.float32

N = 10000
E = 320000
NC = 32
NT = 10240
TD = 128
NW = 32
CH = 128
NCHK = 80
EW = NCHK * CH
EP = NW * EW
BE = 4096
GRID_E = EP // BE
MD = 128
RPS = NT // 16
PH = 5
CPP = NCHK // PH


def _leak(v):
    return jnp.where(v > 0, v, 0.01 * v)


def _dotT(a, w):
    return lax.dot_general(a, w, (((1,), (1,)), ((), ())),
                           precision=lax.Precision.HIGHEST,
                           preferred_element_type=_f32)


_MESH = dict(core_axis_name="c", subcore_axis_name="s")


def _sc_gather(tbl, dstw, srcw):
    @functools.partial(
        pl.kernel,
        out_type=[jax.ShapeDtypeStruct((EP, TD), _f32),
                  jax.ShapeDtypeStruct((EP, TD), _f32)],
        mesh=plsc.VectorSubcoreMesh(**_MESH),
        scratch_types=[
            pltpu.VMEM((CPP, CH), jnp.int32),
            pltpu.VMEM((CPP, CH), jnp.int32),
            pltpu.VMEM((CH, TD), _f32),
            pltpu.VMEM((CH, TD), _f32),
            pltpu.VMEM_SHARED((NT, TD), _f32),
            pltpu.SemaphoreType.DMA,
            pltpu.SemaphoreType.DMA,
        ])
    def k(tbl_hbm, dst_hbm, src_hbm, gd_hbm, gs_hbm,
          idx_d, idx_s, buf_d, buf_s, tbl_sh, sem_d, sem_s):
        sid = lax.axis_index("s")
        wid = sid * 2 + lax.axis_index("c")
        base = wid * EW
        row0 = sid * RPS

        def stg(kk, c):
            pltpu.sync_copy(tbl_hbm.at[pl.ds(row0 + kk * CH, CH)], buf_d)
            pltpu.sync_copy(buf_d, tbl_sh.at[pl.ds(row0 + kk * CH, CH)])
            return c

        lax.fori_loop(0, RPS // CH, stg, 0)
        plsc.subcore_barrier()

        def phase(p, c):
            pltpu.sync_copy(dst_hbm.at[wid * PH + p], idx_d)
            pltpu.sync_copy(src_hbm.at[wid * PH + p], idx_s)

            def body(i, c2):
                cp_d = pltpu.async_copy(tbl_sh.at[idx_d.at[i]], buf_d, sem_d)
                cp_s = pltpu.async_copy(tbl_sh.at[idx_s.at[i]], buf_s, sem_s)
                cp_d.wait()
                cp_s.wait()
                row = base + (p * CPP + i) * CH
                pltpu.sync_copy(buf_d, gd_hbm.at[pl.ds(row, CH)])
                pltpu.sync_copy(buf_s, gs_hbm.at[pl.ds(row, CH)])
                return c2

            return lax.fori_loop(0, CPP, body, c)

        lax.fori_loop(0, PH, phase, 0)

    return k(tbl, dstw, srcw)


def _sc_scatter(msg2, dstw2, zrows):
    @functools.partial(
        pl.kernel,
        out_type=jax.ShapeDtypeStruct((2, NT, MD), _f32),
        mesh=plsc.VectorSubcoreMesh(**_MESH),
        scratch_types=[
            pltpu.VMEM((NCHK, CH), jnp.int32),
            pltpu.VMEM((CH, MD), _f32),
            pltpu.VMEM((CH, MD), _f32),
            pltpu.VMEM_SHARED((NT, MD), _f32),
        ])
    def k(msg_hbm, dst_hbm, z_hbm, out_hbm, idx, bufl, buf2, agg):
        cid = lax.axis_index("c")
        sid = lax.axis_index("s")
        wid = sid * 2 + cid
        row0 = sid * RPS
        pltpu.sync_copy(z_hbm, buf2)

        def z0(kk, c):
            pltpu.sync_copy(buf2, agg.at[pl.ds(row0 + kk * CH, CH)])
            return c

        lax.fori_loop(0, RPS // CH, z0, 0)
        pltpu.sync_copy(dst_hbm.at[wid], idx)
        plsc.subcore_barrier()

        def body(i, c2):
            pltpu.sync_copy(msg_hbm.at[wid * NCHK + i], bufl)
            pltpu.sync_copy(bufl, agg.at[idx.at[i]], add=True)
            return c2

        lax.fori_loop(0, NCHK, body, 0)
        plsc.subcore_barrier()

        def cpout(kk, c):
            pltpu.sync_copy(agg.at[pl.ds(row0 + kk * CH, CH)], buf2)
            pltpu.sync_copy(buf2, out_hbm.at[cid, pl.ds(row0 + kk * CH, CH)])
            return c

        lax.fori_loop(0, RPS // CH, cpout, 0)

    return k(msg2, dstw2, zrows)


def _full(shape):
    return pl.BlockSpec(shape, lambda *_: tuple(0 for _ in shape))


def _prologue(h, x, W, b, g, bb):
    def body(h_ref, x_ref, W_ref, b_ref, g_ref, bb_ref, o_ref):
        z = _dotT(h_ref[...], W_ref[...]) + b_ref[...]
        mu = jnp.mean(z, axis=0, keepdims=True)
        var = jnp.mean(z * z, axis=0, keepdims=True) - mu * mu
        hh = _leak((z - mu) * lax.rsqrt(var + 1e-5) * g_ref[...] + bb_ref[...])
        blk = jnp.concatenate([hh, x_ref[...], jnp.zeros((N, TD - 35), _f32)],
                              axis=1)
        o_ref[...] = jnp.concatenate([blk, jnp.zeros((NT - N, TD), _f32)],
                                     axis=0)

    return pl.pallas_call(
        body,
        out_shape=jax.ShapeDtypeStruct((NT, TD), _f32),
    )(h, x, W, b, g, bb)


def _edge_stats(gd, gs):
    def body(gd_ref, gs_ref, o_ref):
        i = pl.program_id(0)
        gdv = gd_ref[...]
        gsv = gs_ref[...]
        hi = gdv[:, :NC]
        hj = gsv[:, :NC]
        diff = gdv[:, NC:NC + 3] - gsv[:, NC:NC + 3]
        dsq = jnp.sum(diff * diff, axis=1, keepdims=True)
        d = jnp.sqrt(dsq)
        z63 = jnp.zeros((1, 63), _f32)
        s1 = jnp.concatenate([jnp.sum(hi, 0, keepdims=True),
                              jnp.sum(hj, 0, keepdims=True),
                              jnp.sum(d, 0, keepdims=True), z63], axis=1)
        s2 = jnp.concatenate([jnp.sum(hi * hi, 0, keepdims=True),
                              jnp.sum(hj * hj, 0, keepdims=True),
                              jnp.sum(dsq, 0, keepdims=True), z63], axis=1)
        blk = jnp.concatenate([s1, s2], axis=0)

        @pl.when(i == 0)
        def _():
            o_ref[...] = blk

        @pl.when(i != 0)
        def _():
            o_ref[...] = o_ref[...] + blk

    return pl.pallas_call(
        body,
        grid=(GRID_E,),
        in_specs=[pl.BlockSpec((BE, TD), lambda i: (i, 0)),
                  pl.BlockSpec((BE, TD), lambda i: (i, 0))],
        out_specs=pl.BlockSpec((2, 128), lambda i: (0, 0)),
        out_shape=jax.ShapeDtypeStruct((2, 128), _f32),
    )(gd, gs)


def _edge_mlp(gd, gs, st, eg, eb, W1, b1, W2, b2, cW1, cb1, cW2):
    def body(gd_ref, gs_ref, st_ref, eg_ref, eb_ref, W1_ref, b1_ref,
             W2_ref, b2_ref, cW1_ref, cb1_ref, cW2_ref, o_ref):
        inv_e = 1.0 / E
        mu = st_ref[0:1, 0:65] * inv_e
        var = st_ref[1:2, 0:65] * inv_e - mu * mu
        s = eg_ref[...] * lax.rsqrt(var + 1e-5)
        t = eb_ref[...] - mu * s
        gdv = gd_ref[...]
        gsv = gs_ref[...]
        hi = gdv[:, :NC]
        hj = gsv[:, :NC]
        diff = gdv[:, NC:NC + 3] - gsv[:, NC:NC + 3]
        d = jnp.sqrt(jnp.sum(diff * diff, axis=1, keepdims=True))
        m = jnp.concatenate([hi, hj, d], axis=1) * s + t
        y = _leak(_dotT(m, W1_ref[...]) + b1_ref[...])
        m2 = _leak(_dotT(y, W2_ref[...]) + b2_ref[...])
        c1 = _leak(_dotT(m2, cW1_ref[...]) + cb1_ref[...])
        c = _dotT(c1, cW2_ref[...])
        o_ref[...] = jnp.concatenate(
            [m2, diff * c, jnp.zeros((BE, MD - 35), _f32)], axis=1)

    wspecs = [_full(w.shape) for w in
              (st, eg, eb, W1, b1, W2, b2, cW1, cb1, cW2)]
    return pl.pallas_call(
        body,
        grid=(GRID_E,),
        in_specs=[pl.BlockSpec((BE, TD), lambda i: (i, 0)),
                  pl.BlockSpec((BE, TD), lambda i: (i, 0))] + wspecs,
        out_specs=pl.BlockSpec((BE, MD), lambda i: (i, 0)),
        out_shape=jax.ShapeDtypeStruct((EP, MD), _f32),
    )(gd, gs, st, eg, eb, W1, b1, W2, b2, cW1, cb1, cW2)


def _node_update(tprev, parts, W1, b1, g, bb, W2, b2):
    def body(t_ref, p_ref, W1_ref, b1_ref, g_ref, bb_ref, W2_ref, b2_ref,
             o_ref):
        tv = t_ref[...]
        agg = p_ref[0] + p_ref[1]
        hh = tv[:N, 0:NC]
        x_new = tv[:N, NC:NC + 3] + agg[:N, NC:NC + 3]
        z = _dotT(jnp.concatenate([hh, agg[:N, 0:NC]], axis=1),
                  W1_ref[...]) + b1_ref[...]
        mu = jnp.mean(z, axis=0, keepdims=True)
        var = jnp.mean(z * z, axis=0, keepdims=True) - mu * mu
        z = _leak((z - mu) * lax.rsqrt(var + 1e-5) * g_ref[...] + bb_ref[...])
        hn = _dotT(z, W2_ref[...]) + b2_ref[...]
        blk = jnp.concatenate([hn, x_new, jnp.zeros((N, TD - 35), _f32)],
                              axis=1)
        o_ref[...] = jnp.concatenate([blk, jnp.zeros((NT - N, TD), _f32)],
                                     axis=0)

    return pl.pallas_call(
        body,
        out_shape=jax.ShapeDtypeStruct((NT, TD), _f32),
    )(tprev, parts, W1, b1, g, bb, W2, b2)


def _final_proj(t2, W, b):
    def body(t_ref, W_ref, b_ref, o_ref):
        o_ref[...] = _dotT(t_ref[:N, 0:NC], W_ref[...]) + b_ref[...]

    return pl.pallas_call(
        body,
        out_shape=jax.ShapeDtypeStruct((N, 128), _f32),
    )(t2, W, b)


def _jnp_rest(h, x, edge_index, params, use_sc_gather=True, use_sc_scatter=False, gl=(0, 1), sl=(0, 1)):
    p = params
    src = edge_index[0].astype(jnp.int32)
    dst = edge_index[1].astype(jnp.int32)
    pad_ids = (jnp.arange(EP - E, dtype=jnp.int32) % (NT - N)) + N
    dstw = jnp.concatenate([dst, pad_ids]).reshape(NW * PH, CPP, CH)
    srcw = jnp.concatenate([src, pad_ids]).reshape(NW * PH, CPP, CH)
    zrows = jnp.zeros((CH, MD), _f32)

    def bn(v, g, b):
        mu = jnp.mean(v, 0); var = jnp.var(v, 0)
        return (v - mu) / jnp.sqrt(var + 1e-5) * g + b
    hh = _leak(bn(h @ p['ri_W'].T + p['ri_b'], p['ri_g'], p['ri_bb']))
    xx = x
    for l in range(2):
        pre = 'l%d_' % l
        tbl = jnp.zeros((NT, TD), _f32).at[:N, :NC].set(hh).at[:N, NC:NC+3].set(xx)
        if use_sc_gather and l in gl:
            gd, gs = _sc_gather(tbl, dstw, srcw)
        else:
            gd = tbl[dstw.reshape(-1)]
            gs = tbl[srcw.reshape(-1)]
        hi = gd[:E, :NC]; hj = gs[:E, :NC]
        diff = gd[:E, NC:NC+3] - gs[:E, NC:NC+3]
        d = jnp.sqrt(jnp.sum(diff*diff, 1, keepdims=True))
        m = jnp.concatenate([hi, hj, d], 1)
        m = bn(m, p[pre+'ein_g'], p[pre+'ein_b'])
        m = _leak(m @ p[pre+'e_W1'].T + p[pre+'e_b1'])
        m = _leak(m @ p[pre+'e_W2'].T + p[pre+'e_b2'])
        c = _leak(m @ p[pre+'c_W1'].T + p[pre+'c_b1'])
        c = c @ p[pre+'c_W2'].T
        x_ij = diff * c
        msg = jnp.concatenate([m, x_ij, jnp.zeros((E, MD-35), _f32)], 1)
        msgp = jnp.concatenate([msg, jnp.zeros((EP-E, MD), _f32)], 0)
        if use_sc_scatter and l in sl:
            parts = _sc_scatter(msgp.reshape(NW*NCHK, CH, MD),
                                jnp.concatenate([dst, pad_ids]).reshape(NW, NCHK, CH), zrows)
            agg = (parts[0] + parts[1])[:N]
        else:
            agg = jnp.zeros((N, MD), _f32).at[dst].add(msg[:E])
        m_agg = agg[:, :NC]
        xx = xx + agg[:, NC:NC+3]
        z = jnp.concatenate([hh, m_agg], 1) @ p[pre+'n_W1'].T + p[pre+'n_b1']
        hh = _leak(bn(z, p[pre+'n_g'], p[pre+'n_bb'])) @ p[pre+'n_W2'].T + p[pre+'n_b2']
    out = hh @ p['ro_W'].T + p['ro_b']
    return jnp.concatenate([out, xx], 1)


def kernel(h, x, edge_index, params):
    p = params
    src = edge_index[0].astype(jnp.int32)
    dst = edge_index[1].astype(jnp.int32)
    pad_ids = (jnp.arange(EP - E, dtype=jnp.int32) % (NT - N)) + N
    dstw = jnp.concatenate([dst, pad_ids]).reshape(NW * PH, CPP, CH)
    srcw = jnp.concatenate([src, pad_ids]).reshape(NW * PH, CPP, CH)
    zrows = jnp.zeros((CH, MD), _f32)

    def row(v):
        return v.reshape(1, -1)

    tbl = _prologue(h, x, p['ri_W'], row(p['ri_b']), row(p['ri_g']),
                    row(p['ri_bb']))
    for l in range(2):
        pre = 'l%d_' % l
        gd, gs = _sc_gather(tbl, dstw, srcw)
        st = _edge_stats(gd, gs)
        msg = _edge_mlp(gd, gs, st,
                        row(p[pre + 'ein_g']), row(p[pre + 'ein_b']),
                        p[pre + 'e_W1'], row(p[pre + 'e_b1']),
                        p[pre + 'e_W2'], row(p[pre + 'e_b2']),
                        p[pre + 'c_W1'], row(p[pre + 'c_b1']),
                        p[pre + 'c_W2'])
        parts = _sc_scatter(msg.reshape(NW * NCHK, CH, MD),
                            jnp.concatenate([dst, pad_ids]).reshape(NW, NCHK, CH),
                            zrows)
        tbl = _node_update(tbl, parts,
                           p[pre + 'n_W1'], row(p[pre + 'n_b1']),
                           row(p[pre + 'n_g']), row(p[pre + 'n_bb']),
                           p[pre + 'n_W2'], row(p[pre + 'n_b2']))
    out = _final_proj(tbl, p['ro_W'], row(p['ro_b']))
    return jnp.concatenate([out, tbl[:N, NC:NC + 3]], axis=1)
